# baseline (speedup 1.0000x reference)
import functools

import jax
import jax.numpy as jnp
from jax import lax
from jax.experimental import pallas as pl
from jax.experimental.pallas import tpu as pltpu
from jax.experimental.pallas import tpu_sc as plsc

NC = 2
NS = 16
NW = NC * NS
CHUNK = 128
LANES = 16


def _sc_update(B, D):
    mesh = plsc.VectorSubcoreMesh(core_axis_name="c", subcore_axis_name="s")
    maxch = B // CHUNK

    @functools.partial(
        pl.kernel,
        out_type=(),
        mesh=mesh,
        compiler_params=pltpu.CompilerParams(
            needs_layout_passes=False, use_tc_tiling_on_sc=False),
        scratch_types=[
            pltpu.VMEM((CHUNK,), jnp.int32),
            pltpu.VMEM((CHUNK,), jnp.int32),
            pltpu.VMEM((CHUNK, D), jnp.float32),
            pltpu.VMEM((CHUNK, D), jnp.float32),
            pltpu.VMEM((LANES,), jnp.float32),
            pltpu.VMEM((LANES,), jnp.int32),
            pltpu.SemaphoreType.DMA,
        ],
    )
    def body(idx_hbm, pos_hbm, cnt_hbm, mem_hbm, mom_hbm, table,
             idxv, posv, oldv, newv, momv, cntv, sem):
        c = lax.axis_index("c")
        s = lax.axis_index("s")
        w = s * NC + c
        pltpu.sync_copy(mom_hbm, momv)
        pltpu.sync_copy(cnt_hbm.at[w], cntv)
        mval = momv[...]
        one_m = 1.0 - mval
        cnt = jnp.max(cntv[...])

        @pl.loop(0, maxch)
        def _chunk(j):
            @pl.when(j * CHUNK < cnt)
            def _():
                pltpu.sync_copy(idx_hbm.at[w, pl.ds(j * CHUNK, CHUNK)], idxv)
                pltpu.sync_copy(pos_hbm.at[w, pl.ds(j * CHUNK, CHUNK)], posv)
                pltpu.async_copy(table.at[idxv], oldv, sem).wait()
                pltpu.async_copy(mem_hbm.at[posv], newv, sem).wait()

                @pl.loop(0, CHUNK)
                def _row(r):
                    acc = jnp.zeros((LANES,), jnp.float32)
                    for k in range(D // LANES):
                        o = oldv[r, pl.ds(k * LANES, LANES)]
                        n = newv[r, pl.ds(k * LANES, LANES)]
                        u = o * one_m + n * mval
                        oldv[r, pl.ds(k * LANES, LANES)] = u
                        acc = acc + u * u
                    ssum = jnp.sum(acc)
                    sv = lax.broadcast_in_dim(ssum, (LANES,), ())
                    iv = plsc.bitcast(sv, jnp.int32)
                    iv = jnp.int32(0x5F3759DF) - lax.shift_right_logical(iv, 1)
                    y = plsc.bitcast(iv, jnp.float32)
                    for _ in range(3):
                        y = y * (1.5 - 0.5 * sv * y * y)
                    for k in range(D // LANES):
                        oldv[r, pl.ds(k * LANES, LANES)] = (
                            oldv[r, pl.ds(k * LANES, LANES)] * y)

                pltpu.async_copy(oldv, table.at[idxv], sem).wait()

    return body


def kernel(mem, momentum, ind, time, memory):
    mem2 = mem.reshape(mem.shape[0], -1)
    B, D = mem2.shape
    L = memory.shape[0]
    ind32 = ind.astype(jnp.int32)
    R = L // NW

    iota = jnp.arange(B, dtype=jnp.int32)
    pos_table = jnp.zeros((L,), jnp.int32).at[ind32].set(iota)
    winner_pos = pos_table[ind32]
    keep = winner_pos == iota

    owner = ind32 // R
    order = jnp.argsort(jnp.where(keep, owner, NW), stable=True).astype(
        jnp.int32)
    ind_s = ind32[order]
    counts = jnp.zeros((NW,), jnp.int32).at[owner].add(keep.astype(jnp.int32))
    offs = jnp.cumsum(counts).astype(jnp.int32) - counts

    t = jnp.arange(B, dtype=jnp.int32)
    gidx = offs[:, None] + jnp.minimum(
        t[None, :], jnp.maximum(counts[:, None] - 1, 0))
    gidx = jnp.clip(gidx, 0, B - 1)
    idx_m = ind_s[gidx]
    pos_m = order[gidx]
    cnt16 = jnp.broadcast_to(counts[:, None], (NW, LANES))
    mom16 = jnp.full((LANES,), momentum, jnp.float32)

    table_ref = jax.new_ref(memory)
    _sc_update(B, D)(idx_m, pos_m, cnt16, mem2, mom16, table_ref)
    return jax.freeze(table_ref)


# speedup vs baseline: 9.7455x; 9.7455x over previous
import functools

import jax
import jax.numpy as jnp
from jax import lax
from jax.experimental import pallas as pl
from jax.experimental.pallas import tpu as pltpu
from jax.experimental.pallas import tpu_sc as plsc

NC = 2
NS = 16
NW = NC * NS
CHUNK = 128
LANES = 16
MAXCH = 129


def _bc(x, dtype):
    return lax.broadcast_in_dim(x.astype(dtype), (LANES,), ())


def _sc_update(B, D):
    mesh = plsc.VectorSubcoreMesh(core_axis_name="c", subcore_axis_name="s")

    @functools.partial(
        pl.kernel,
        out_type=(),
        mesh=mesh,
        compiler_params=pltpu.CompilerParams(
            needs_layout_passes=False, use_tc_tiling_on_sc=False),
        scratch_types=[
            pltpu.VMEM((CHUNK,), jnp.int32),
            pltpu.VMEM((CHUNK,), jnp.int32),
            pltpu.VMEM((CHUNK, D), jnp.float32),
            pltpu.VMEM((CHUNK, D), jnp.float32),
            pltpu.VMEM((LANES,), jnp.float32),
            pltpu.VMEM((LANES,), jnp.int32),
            pltpu.VMEM((LANES,), jnp.int32),
            pltpu.SemaphoreType.DMA,
        ],
    )
    def body(idx_hbm, pos_hbm, off_hbm, cnt_hbm, mem_hbm, mom_hbm, table,
             idxv, posv, oldv, newv, momv, offv, cntv, sem):
        c = lax.axis_index("c")
        s = lax.axis_index("s")
        w = s * NC + c
        pltpu.sync_copy(mom_hbm, momv)
        pltpu.sync_copy(off_hbm.at[w], offv)
        pltpu.sync_copy(cnt_hbm.at[w], cntv)
        mval = momv[...]
        one_m = 1.0 - mval
        off = jnp.max(offv[...])
        cnt = jnp.max(cntv[...])
        end = off + cnt
        start = off & -8
        iota16 = lax.iota(jnp.int32, LANES)

        @pl.loop(0, MAXCH)
        def _chunk(j):
            base = pl.multiple_of(start + j * CHUNK, 8)

            @pl.when((cnt > 0) & (base < end))
            def _():
                pltpu.sync_copy(idx_hbm.at[pl.ds(base, CHUNK)], idxv)
                pltpu.sync_copy(pos_hbm.at[pl.ds(base, CHUNK)], posv)
                fv = jnp.maximum(off - base, 0)
                sel0 = iota16 == _bc(fv, jnp.int32)
                v0 = idxv[pl.ds(0, LANES)]
                p0 = posv[pl.ds(0, LANES)]
                fb_i = _bc(jnp.sum(jnp.where(sel0, v0, 0)), jnp.int32)
                fb_p = _bc(jnp.sum(jnp.where(sel0, p0, 0)), jnp.int32)
                for k in range(CHUNK // LANES):
                    g = iota16 + _bc(base + k * LANES, jnp.int32)
                    valid = (g >= _bc(off, jnp.int32)) & (
                        g < _bc(end, jnp.int32))
                    vk = idxv[pl.ds(k * LANES, LANES)]
                    pk = posv[pl.ds(k * LANES, LANES)]
                    idxv[pl.ds(k * LANES, LANES)] = jnp.where(valid, vk, fb_i)
                    posv[pl.ds(k * LANES, LANES)] = jnp.where(valid, pk, fb_p)

                pltpu.async_copy(table.at[idxv], oldv, sem).wait()
                pltpu.async_copy(mem_hbm.at[posv], newv, sem).wait()

                @pl.loop(0, CHUNK)
                def _row(r):
                    acc = jnp.zeros((LANES,), jnp.float32)
                    for k in range(D // LANES):
                        o = oldv[r, pl.ds(k * LANES, LANES)]
                        n = newv[r, pl.ds(k * LANES, LANES)]
                        u = o * one_m + n * mval
                        oldv[r, pl.ds(k * LANES, LANES)] = u
                        acc = acc + u * u
                    ssum = jnp.sum(acc)
                    sv = lax.broadcast_in_dim(ssum, (LANES,), ())
                    iv = plsc.bitcast(sv, jnp.int32)
                    iv = jnp.int32(0x5F3759DF) - lax.shift_right_logical(iv, 1)
                    y = plsc.bitcast(iv, jnp.float32)
                    for _ in range(3):
                        y = y * (1.5 - 0.5 * sv * y * y)
                    for k in range(D // LANES):
                        oldv[r, pl.ds(k * LANES, LANES)] = (
                            oldv[r, pl.ds(k * LANES, LANES)] * y)

                pltpu.async_copy(oldv, table.at[idxv], sem).wait()

    return body


def kernel(mem, momentum, ind, time, memory):
    mem2 = mem.reshape(mem.shape[0], -1)
    B, D = mem2.shape
    L = memory.shape[0]
    ind32 = ind.astype(jnp.int32)
    R = L // NW

    iota = jnp.arange(B, dtype=jnp.int32)
    pos_table = jnp.zeros((L,), jnp.int32).at[ind32].set(iota)
    winner_pos = pos_table[ind32]
    keep = winner_pos == iota

    owner = ind32 // R
    order = jnp.argsort(jnp.where(keep, owner, NW), stable=True).astype(
        jnp.int32)
    ind_s = ind32[order]
    counts = jnp.zeros((NW,), jnp.int32).at[owner].add(keep.astype(jnp.int32))
    offs = jnp.cumsum(counts).astype(jnp.int32) - counts

    pad_i = jnp.broadcast_to(ind_s[-1:], (CHUNK,))
    pad_p = jnp.broadcast_to(order[-1:], (CHUNK,))
    ind_p = jnp.concatenate([ind_s, pad_i])
    pos_p = jnp.concatenate([order, pad_p])
    off16 = jnp.broadcast_to(offs[:, None], (NW, LANES))
    cnt16 = jnp.broadcast_to(counts[:, None], (NW, LANES))
    mom16 = jnp.full((LANES,), momentum, jnp.float32)

    table_ref = jax.new_ref(memory)
    _sc_update(B, D)(ind_p, pos_p, off16, cnt16, mem2, mom16, table_ref)
    return jax.freeze(table_ref)
